# baseline (speedup 1.0000x reference)
import jax
import jax.numpy as jnp
from jax.experimental import pallas as pl
from jax.experimental.pallas import tpu as pltpu

H = 2048
W = 2048
N = H * W


def _kern(si_ref, sf_ref, cpm_hbm, mask_hbm,
          cpm_out, mask_out, energy_out, delta_out, accept_out,
          maskv, T, tATs, tB2s, c0rows, c1rows, srows, mrows, sems):
    cpm_cps = []
    for k in range(8):
        cp = pltpu.make_async_copy(
            cpm_hbm.at[:, pl.ds(k * 256, 256), :],
            cpm_out.at[:, pl.ds(k * 256, 256), :], sems.at[18 + k])
        cp.start()
        cpm_cps.append(cp)

    cpin = pltpu.make_async_copy(mask_hbm, maskv, sems.at[17])
    cpin.start()
    cpin.wait()

    mask_cp = pltpu.make_async_copy(maskv, mask_out, sems.at[26])
    mask_cp.start()

    s_sum = sf_ref[3]

    p2 = maskv[:, :] / s_sum
    pt = jnp.transpose(p2.reshape(N // 128, 128))
    T[:, :, :] = pt.reshape(128, 256, 128)

    def scan_body(j, _):
        T[pl.ds(j, 1)] = T[pl.ds(j - 1, 1)] + T[pl.ds(j, 1)]
        return 0

    jax.lax.fori_loop(1, 128, scan_body, 0)

    tATs[:, :] = jnp.transpose(T[127])

    def scanB_step(l, _):
        tATs[pl.ds(l, 1), :] = tATs[pl.ds(l - 1, 1), :] + tATs[pl.ds(l, 1), :]
        return 0

    jax.lax.fori_loop(1, 128, scanB_step, 0)
    totB = tATs[127:128, :]

    tB2s[:, :] = jnp.transpose(totB.reshape(2, 128))

    def scanC_step(l, _):
        tB2s[pl.ds(l, 1), :] = tB2s[pl.ds(l - 1, 1), :] + tB2s[pl.ds(l, 1), :]
        return 0

    jax.lax.fori_loop(1, 128, scanC_step, 0)
    totC = tB2s[127:128, :]
    lane2 = jax.lax.broadcasted_iota(jnp.int32, (1, 2), 1)
    exclD = jnp.where(lane2 == 0, 0.0, totC[0, 0])
    cum256_T = tB2s[:, :] + exclD

    cum256 = jnp.transpose(cum256_T).reshape(1, 256)
    lane256 = jax.lax.broadcasted_iota(jnp.int32, (1, 256), 1)
    excl256 = jnp.where(lane256 == 0, 0.0, pltpu.roll(cum256, 1, 1))

    scanBk = jnp.transpose(tATs[:, :])
    cum32768 = scanBk + jnp.transpose(excl256)

    rolled = pltpu.roll(cum32768, 1, 1)
    lastcol = cum32768[:, 127:128]
    sub256 = jax.lax.broadcasted_iota(jnp.int32, (256, 1), 0)
    prevlast = jnp.where(sub256 == 0, 0.0, pltpu.roll(lastcol, 1, 0))
    lane128 = jax.lax.broadcasted_iota(jnp.int32, (256, 128), 1)
    CA = jnp.where(lane128 == 0, prevlast, rolled)

    def off_body(j, _):
        T[pl.ds(j, 1)] = T[pl.ds(j, 1)] + CA[None, :, :]
        return 0

    jax.lax.fori_loop(0, 128, off_body, 0)

    sub2 = jax.lax.broadcasted_iota(jnp.int32, (256, 128), 0)

    def probe(m):
        j = jax.lax.rem(m, 128)
        i = jax.lax.div(m, 128)
        k = jax.lax.div(i, 128)
        l = jax.lax.rem(i, 128)
        slab = T[pl.ds(j, 1)][0]
        return jnp.sum(jnp.where((sub2 == k) & (lane128 == l), slab, 0.0))

    u = sf_ref[0]
    s_p = T[127][255:256, 127:128][0, 0]
    r = s_p * (1.0 - u)

    def bs_body(_, st):
        low, high = st
        mid = low + jax.lax.div(high - low, 2)
        le = r <= probe(mid)
        return (jnp.where(le, low, mid), jnp.where(le, mid, high))

    _, ind = jax.lax.fori_loop(0, 23, bs_body,
                               (jnp.int32(0), jnp.int32(N)))

    sx = jax.lax.div(ind, W)
    sy = jax.lax.rem(ind, W)
    nx = (sx + si_ref[0] + H) % H
    ny = (sy + si_ref[1] + W) % W

    u_acc = sf_ref[1]
    orig_e = sf_ref[2]
    temperature = sf_ref[4]

    r_m2 = (nx + H - 2) % H
    r_m1 = (nx + H - 1) % H
    r_p1 = (nx + 1) % H
    r_p2 = (nx + 2) % H
    c_m2 = (ny + W - 2) % W
    c_m1 = (ny + W - 1) % W
    c_p1 = (ny + 1) % W
    c_p2 = (ny + 2) % W

    cps = []
    for k, rr in enumerate((r_m2, r_m1, nx, r_p1, r_p2)):
        cp = pltpu.make_async_copy(
            cpm_hbm.at[0, pl.ds(rr, 1), :], c0rows.at[pl.ds(k, 1), :], sems.at[k])
        cp.start()
        cps.append(cp)
    for k, rr in enumerate((r_m1, nx, r_p1)):
        cp = pltpu.make_async_copy(
            cpm_hbm.at[1, pl.ds(rr, 1), :], c1rows.at[pl.ds(k, 1), :], sems.at[5 + k])
        cp.start()
        cps.append(cp)
    for k in range(2):
        cp = pltpu.make_async_copy(
            cpm_hbm.at[k, pl.ds(sx, 1), :], srows.at[pl.ds(k, 1), :], sems.at[8 + k])
        cp.start()
        cps.append(cp)
    for cp in cps:
        cp.wait()

    lane = jax.lax.broadcasted_iota(jnp.int32, (1, W), 1)

    def get(rowvec, col):
        return jnp.sum(jnp.where(lane == col, rowvec, 0.0))

    c0_m2 = c0rows[0:1, :]
    c0_m1 = c0rows[1:2, :]
    c0_0 = c0rows[2:3, :]
    c0_p1 = c0rows[3:4, :]
    c0_p2 = c0rows[4:5, :]
    c1_m1 = c1rows[0:1, :]
    c1_0 = c1rows[1:2, :]
    c1_p1 = c1rows[2:3, :]

    src0 = get(srows[0:1, :], sy)
    src1 = get(srows[1:2, :], sy)
    old0 = get(c0_0, ny)
    old1 = get(c1_0, ny)

    n0 = (get(c0_p1, ny), get(c0_m1, ny), get(c0_0, c_p1), get(c0_0, c_m1))
    n1 = (get(c1_p1, ny), get(c1_m1, ny), get(c1_0, c_p1), get(c1_0, c_m1))

    def local_energy(v0, v1):
        e = jnp.float32(0.0)
        for k in range(4):
            e = e + ((v0 - n0[k]) ** 2 + (v1 - n1[k]) ** 2)
        return e

    delta = (1.0 / temperature) * (local_energy(src0, src1)
                                   - local_energy(old0, old1))

    p_acc = jnp.clip(jnp.exp(-delta), 0.0, 1.0)
    accept = (delta >= 0) | (u_acc < p_acc)

    energy_out[0] = jnp.where(accept, orig_e + delta, orig_e)
    delta_out[0] = delta
    accept_out[0] = accept.astype(jnp.int32)

    hit = (lane == ny) & accept
    new0 = jnp.where(hit, src0, c0_0)
    new1 = jnp.where(hit, src1, c1_0)
    for cp in cpm_cps:
        cp.wait()
    mask_cp.wait()
    w0 = pltpu.make_async_copy(c0rows.at[pl.ds(5, 1), :],
                               cpm_out.at[0, pl.ds(nx, 1), :], sems.at[13])
    c0rows[5:6, :] = new0
    w0.start()
    w1 = pltpu.make_async_copy(c1rows.at[pl.ds(3, 1), :],
                               cpm_out.at[1, pl.ds(nx, 1), :], sems.at[14])
    c1rows[3:4, :] = new1
    w1.start()

    def is_boundary(selfv, q1, q2, q3, q4):
        b = (jnp.abs(q1 - selfv) > 1e-6) | (jnp.abs(q2 - selfv) > 1e-6) \
            | (jnp.abs(q3 - selfv) > 1e-6) | (jnp.abs(q4 - selfv) > 1e-6)
        return b.astype(jnp.float32)

    vA = is_boundary(get(new0, ny),
                     get(c0_p1, ny), get(c0_m1, ny),
                     get(new0, c_p1), get(new0, c_m1))
    vB = is_boundary(get(c0_p1, ny),
                     get(c0_p2, ny), get(new0, ny),
                     get(c0_p1, c_p1), get(c0_p1, c_m1))
    vC = is_boundary(get(c0_m1, ny),
                     get(new0, ny), get(c0_m2, ny),
                     get(c0_m1, c_p1), get(c0_m1, c_m1))
    vD = is_boundary(get(new0, c_p1),
                     get(c0_p1, c_p1), get(c0_m1, c_p1),
                     get(new0, c_p2), get(new0, ny))
    vE = is_boundary(get(new0, c_m1),
                     get(c0_p1, c_m1), get(c0_m1, c_m1),
                     get(new0, ny), get(new0, c_m2))

    def maskrow(rr):
        return maskv[pl.ds(rr, 1), :]

    m_m1 = jnp.where(lane == ny, vC, maskrow(r_m1))
    m_0 = maskrow(nx)
    m_0 = jnp.where(lane == ny, vA, m_0)
    m_0 = jnp.where(lane == c_p1, vD, m_0)
    m_0 = jnp.where(lane == c_m1, vE, m_0)
    m_p1 = jnp.where(lane == ny, vB, maskrow(r_p1))

    wm = []
    for k, (rr, vec) in enumerate(((r_m1, m_m1), (nx, m_0), (r_p1, m_p1))):
        mrows[k:k + 1, :] = vec
        cp = pltpu.make_async_copy(mrows.at[pl.ds(k, 1), :],
                                   mask_out.at[pl.ds(rr, 1), :], sems.at[15 + k])
        cp.start()
        wm.append(cp)
    w0.wait()
    w1.wait()
    for cp in wm:
        cp.wait()


def kernel(cpm, original_energy, boundary_mask, temperature, seed):
    key = jax.random.key(seed)
    key, use_key = jax.random.split(key)
    u_choice = jax.random.uniform(use_key, (), jnp.float32)
    key, use_key = jax.random.split(key)
    i = jax.random.randint(use_key, (), 0, 4)
    dx = jnp.array([1, -1, 0, 0], jnp.int32)
    dy = jnp.array([0, 0, 1, -1], jnp.int32)
    key, acc_key = jax.random.split(key)
    u_acc = jax.random.uniform(acc_key, (), jnp.float32)
    s_sum = boundary_mask.sum()

    si = jnp.stack([dx[i], dy[i]]).astype(jnp.int32)
    sf = jnp.stack([u_choice, u_acc, original_energy, s_sum,
                    temperature]).astype(jnp.float32)

    out_shape = [
        jax.ShapeDtypeStruct((2, H, W), jnp.float32),
        jax.ShapeDtypeStruct((H, W), jnp.float32),
        jax.ShapeDtypeStruct((1,), jnp.float32),
        jax.ShapeDtypeStruct((1,), jnp.float32),
        jax.ShapeDtypeStruct((1,), jnp.int32),
    ]
    cpm_out, mask_out, energy, delta, accept = pl.pallas_call(
        _kern,
        in_specs=[
            pl.BlockSpec(memory_space=pltpu.SMEM),
            pl.BlockSpec(memory_space=pltpu.SMEM),
            pl.BlockSpec(memory_space=pl.ANY),
            pl.BlockSpec(memory_space=pl.ANY),
        ],
        out_specs=[
            pl.BlockSpec(memory_space=pl.ANY),
            pl.BlockSpec(memory_space=pl.ANY),
            pl.BlockSpec(memory_space=pltpu.SMEM),
            pl.BlockSpec(memory_space=pltpu.SMEM),
            pl.BlockSpec(memory_space=pltpu.SMEM),
        ],
        out_shape=out_shape,
        scratch_shapes=[
            pltpu.VMEM((H, W), jnp.float32),
            pltpu.VMEM((128, 256, 128), jnp.float32),
            pltpu.VMEM((128, 256), jnp.float32),
            pltpu.VMEM((128, 2), jnp.float32),
            pltpu.VMEM((6, W), jnp.float32),
            pltpu.VMEM((4, W), jnp.float32),
            pltpu.VMEM((2, W), jnp.float32),
            pltpu.VMEM((3, W), jnp.float32),
            pltpu.SemaphoreType.DMA((27,)),
        ],
    )(si, sf, cpm, boundary_mask)
    return (cpm_out, jnp.reshape(energy, ()), mask_out,
            jnp.reshape(delta, ()), jnp.reshape(accept, ()))


# speedup vs baseline: 9.4545x; 9.4545x over previous
import jax
import jax.numpy as jnp
from jax.experimental import pallas as pl
from jax.experimental.pallas import tpu as pltpu

H = 2048
W = 2048
N = H * W


def _kern(si_ref, sf_ref, cpm_hbm, mask_hbm,
          cpm_out, mask_out, energy_out, delta_out, accept_out,
          maskv, T, tATs, tB2s, c0rows, c1rows, srows, mrows, sems):
    cpin = pltpu.make_async_copy(mask_hbm, maskv, sems.at[17])
    cpin.start()
    cpin.wait()

    s_sum = sf_ref[3]

    p2 = maskv[:, :] / s_sum
    pt = jnp.transpose(p2.reshape(N // 128, 128))
    T[:, :, :] = pt.reshape(128, 256, 128)

    def scan_body(j, _):
        T[pl.ds(j, 1)] = T[pl.ds(j - 1, 1)] + T[pl.ds(j, 1)]
        return 0

    jax.lax.fori_loop(1, 128, scan_body, 0)

    tATs[:, :] = jnp.transpose(T[127])

    def scanB_step(l, _):
        tATs[pl.ds(l, 1), :] = tATs[pl.ds(l - 1, 1), :] + tATs[pl.ds(l, 1), :]
        return 0

    jax.lax.fori_loop(1, 128, scanB_step, 0)
    totB = tATs[127:128, :]

    tB2s[:, :] = jnp.transpose(totB.reshape(2, 128))

    def scanC_step(l, _):
        tB2s[pl.ds(l, 1), :] = tB2s[pl.ds(l - 1, 1), :] + tB2s[pl.ds(l, 1), :]
        return 0

    jax.lax.fori_loop(1, 128, scanC_step, 0)
    totC = tB2s[127:128, :]
    lane2 = jax.lax.broadcasted_iota(jnp.int32, (1, 2), 1)
    exclD = jnp.where(lane2 == 0, 0.0, totC[0, 0])
    cum256_T = tB2s[:, :] + exclD

    cum256 = jnp.transpose(cum256_T).reshape(1, 256)
    lane256 = jax.lax.broadcasted_iota(jnp.int32, (1, 256), 1)
    excl256 = jnp.where(lane256 == 0, 0.0, pltpu.roll(cum256, 1, 1))

    scanBk = jnp.transpose(tATs[:, :])
    cum32768 = scanBk + jnp.transpose(excl256)

    rolled = pltpu.roll(cum32768, 1, 1)
    lastcol = cum32768[:, 127:128]
    sub256 = jax.lax.broadcasted_iota(jnp.int32, (256, 1), 0)
    prevlast = jnp.where(sub256 == 0, 0.0, pltpu.roll(lastcol, 1, 0))
    lane128 = jax.lax.broadcasted_iota(jnp.int32, (256, 128), 1)
    CA = jnp.where(lane128 == 0, prevlast, rolled)

    def off_body(j, _):
        T[pl.ds(j, 1)] = T[pl.ds(j, 1)] + CA[None, :, :]
        return 0

    jax.lax.fori_loop(0, 128, off_body, 0)

    sub2 = jax.lax.broadcasted_iota(jnp.int32, (256, 128), 0)

    def probe(m):
        j = jax.lax.rem(m, 128)
        i = jax.lax.div(m, 128)
        k = jax.lax.div(i, 128)
        l = jax.lax.rem(i, 128)
        slab = T[pl.ds(j, 1)][0]
        return jnp.sum(jnp.where((sub2 == k) & (lane128 == l), slab, 0.0))

    u = sf_ref[0]
    s_p = T[127][255:256, 127:128][0, 0]
    r = s_p * (1.0 - u)

    def bs_body(_, st):
        low, high = st
        mid = low + jax.lax.div(high - low, 2)
        le = r <= probe(mid)
        return (jnp.where(le, low, mid), jnp.where(le, mid, high))

    _, ind = jax.lax.fori_loop(0, 23, bs_body,
                               (jnp.int32(0), jnp.int32(N)))

    sx = jax.lax.div(ind, W)
    sy = jax.lax.rem(ind, W)
    nx = (sx + si_ref[0] + H) % H
    ny = (sy + si_ref[1] + W) % W

    u_acc = sf_ref[1]
    orig_e = sf_ref[2]
    temperature = sf_ref[4]

    r_m2 = (nx + H - 2) % H
    r_m1 = (nx + H - 1) % H
    r_p1 = (nx + 1) % H
    r_p2 = (nx + 2) % H
    c_m2 = (ny + W - 2) % W
    c_m1 = (ny + W - 1) % W
    c_p1 = (ny + 1) % W
    c_p2 = (ny + 2) % W

    cps = []
    for k, rr in enumerate((r_m2, r_m1, nx, r_p1, r_p2)):
        cp = pltpu.make_async_copy(
            cpm_hbm.at[0, pl.ds(rr, 1), :], c0rows.at[pl.ds(k, 1), :], sems.at[k])
        cp.start()
        cps.append(cp)
    for k, rr in enumerate((r_m1, nx, r_p1)):
        cp = pltpu.make_async_copy(
            cpm_hbm.at[1, pl.ds(rr, 1), :], c1rows.at[pl.ds(k, 1), :], sems.at[5 + k])
        cp.start()
        cps.append(cp)
    for k in range(2):
        cp = pltpu.make_async_copy(
            cpm_hbm.at[k, pl.ds(sx, 1), :], srows.at[pl.ds(k, 1), :], sems.at[8 + k])
        cp.start()
        cps.append(cp)
    for cp in cps:
        cp.wait()

    lane = jax.lax.broadcasted_iota(jnp.int32, (1, W), 1)

    def get(rowvec, col):
        return jnp.sum(jnp.where(lane == col, rowvec, 0.0))

    c0_m2 = c0rows[0:1, :]
    c0_m1 = c0rows[1:2, :]
    c0_0 = c0rows[2:3, :]
    c0_p1 = c0rows[3:4, :]
    c0_p2 = c0rows[4:5, :]
    c1_m1 = c1rows[0:1, :]
    c1_0 = c1rows[1:2, :]
    c1_p1 = c1rows[2:3, :]

    src0 = get(srows[0:1, :], sy)
    src1 = get(srows[1:2, :], sy)
    old0 = get(c0_0, ny)
    old1 = get(c1_0, ny)

    n0 = (get(c0_p1, ny), get(c0_m1, ny), get(c0_0, c_p1), get(c0_0, c_m1))
    n1 = (get(c1_p1, ny), get(c1_m1, ny), get(c1_0, c_p1), get(c1_0, c_m1))

    def local_energy(v0, v1):
        e = jnp.float32(0.0)
        for k in range(4):
            e = e + ((v0 - n0[k]) ** 2 + (v1 - n1[k]) ** 2)
        return e

    delta = (1.0 / temperature) * (local_energy(src0, src1)
                                   - local_energy(old0, old1))

    p_acc = jnp.clip(jnp.exp(-delta), 0.0, 1.0)
    accept = (delta >= 0) | (u_acc < p_acc)

    energy_out[0] = jnp.where(accept, orig_e + delta, orig_e)
    delta_out[0] = delta
    accept_out[0] = accept.astype(jnp.int32)

    hit = (lane == ny) & accept
    new0 = jnp.where(hit, src0, c0_0)
    new1 = jnp.where(hit, src1, c1_0)
    w0 = pltpu.make_async_copy(c0rows.at[pl.ds(5, 1), :],
                               cpm_out.at[0, pl.ds(nx, 1), :], sems.at[13])
    c0rows[5:6, :] = new0
    w0.start()
    w1 = pltpu.make_async_copy(c1rows.at[pl.ds(3, 1), :],
                               cpm_out.at[1, pl.ds(nx, 1), :], sems.at[14])
    c1rows[3:4, :] = new1
    w1.start()

    def is_boundary(selfv, q1, q2, q3, q4):
        b = (jnp.abs(q1 - selfv) > 1e-6) | (jnp.abs(q2 - selfv) > 1e-6) \
            | (jnp.abs(q3 - selfv) > 1e-6) | (jnp.abs(q4 - selfv) > 1e-6)
        return b.astype(jnp.float32)

    vA = is_boundary(get(new0, ny),
                     get(c0_p1, ny), get(c0_m1, ny),
                     get(new0, c_p1), get(new0, c_m1))
    vB = is_boundary(get(c0_p1, ny),
                     get(c0_p2, ny), get(new0, ny),
                     get(c0_p1, c_p1), get(c0_p1, c_m1))
    vC = is_boundary(get(c0_m1, ny),
                     get(new0, ny), get(c0_m2, ny),
                     get(c0_m1, c_p1), get(c0_m1, c_m1))
    vD = is_boundary(get(new0, c_p1),
                     get(c0_p1, c_p1), get(c0_m1, c_p1),
                     get(new0, c_p2), get(new0, ny))
    vE = is_boundary(get(new0, c_m1),
                     get(c0_p1, c_m1), get(c0_m1, c_m1),
                     get(new0, ny), get(new0, c_m2))

    def maskrow(rr):
        return maskv[pl.ds(rr, 1), :]

    m_m1 = jnp.where(lane == ny, vC, maskrow(r_m1))
    m_0 = maskrow(nx)
    m_0 = jnp.where(lane == ny, vA, m_0)
    m_0 = jnp.where(lane == c_p1, vD, m_0)
    m_0 = jnp.where(lane == c_m1, vE, m_0)
    m_p1 = jnp.where(lane == ny, vB, maskrow(r_p1))

    wm = []
    for k, (rr, vec) in enumerate(((r_m1, m_m1), (nx, m_0), (r_p1, m_p1))):
        mrows[k:k + 1, :] = vec
        cp = pltpu.make_async_copy(mrows.at[pl.ds(k, 1), :],
                                   mask_out.at[pl.ds(rr, 1), :], sems.at[15 + k])
        cp.start()
        wm.append(cp)
    w0.wait()
    w1.wait()
    for cp in wm:
        cp.wait()


def kernel(cpm, original_energy, boundary_mask, temperature, seed):
    key = jax.random.key(seed)
    key, use_key = jax.random.split(key)
    u_choice = jax.random.uniform(use_key, (), jnp.float32)
    key, use_key = jax.random.split(key)
    i = jax.random.randint(use_key, (), 0, 4)
    dx = jnp.array([1, -1, 0, 0], jnp.int32)
    dy = jnp.array([0, 0, 1, -1], jnp.int32)
    key, acc_key = jax.random.split(key)
    u_acc = jax.random.uniform(acc_key, (), jnp.float32)
    s_sum = boundary_mask.sum()

    si = jnp.stack([dx[i], dy[i]]).astype(jnp.int32)
    sf = jnp.stack([u_choice, u_acc, original_energy, s_sum,
                    temperature]).astype(jnp.float32)

    out_shape = [
        jax.ShapeDtypeStruct((2, H, W), jnp.float32),
        jax.ShapeDtypeStruct((H, W), jnp.float32),
        jax.ShapeDtypeStruct((1,), jnp.float32),
        jax.ShapeDtypeStruct((1,), jnp.float32),
        jax.ShapeDtypeStruct((1,), jnp.int32),
    ]
    cpm_out, mask_out, energy, delta, accept = pl.pallas_call(
        _kern,
        in_specs=[
            pl.BlockSpec(memory_space=pltpu.SMEM),
            pl.BlockSpec(memory_space=pltpu.SMEM),
            pl.BlockSpec(memory_space=pl.ANY),
            pl.BlockSpec(memory_space=pl.ANY),
        ],
        out_specs=[
            pl.BlockSpec(memory_space=pl.ANY),
            pl.BlockSpec(memory_space=pl.ANY),
            pl.BlockSpec(memory_space=pltpu.SMEM),
            pl.BlockSpec(memory_space=pltpu.SMEM),
            pl.BlockSpec(memory_space=pltpu.SMEM),
        ],
        out_shape=out_shape,
        input_output_aliases={2: 0, 3: 1},
        scratch_shapes=[
            pltpu.VMEM((H, W), jnp.float32),
            pltpu.VMEM((128, 256, 128), jnp.float32),
            pltpu.VMEM((128, 256), jnp.float32),
            pltpu.VMEM((128, 2), jnp.float32),
            pltpu.VMEM((6, W), jnp.float32),
            pltpu.VMEM((4, W), jnp.float32),
            pltpu.VMEM((2, W), jnp.float32),
            pltpu.VMEM((3, W), jnp.float32),
            pltpu.SemaphoreType.DMA((27,)),
        ],
    )(si, sf, cpm, boundary_mask)
    return (cpm_out, jnp.reshape(energy, ()), mask_out,
            jnp.reshape(delta, ()), jnp.reshape(accept, ()))
